# baseline (speedup 1.0000x reference)
import jax
import jax.numpy as jnp
from jax import lax
from jax.experimental import pallas as pl
from jax.experimental.pallas import tpu as pltpu
from jax.experimental.pallas import tpu_sc as plsc

N = 100000
NP = 102400
PR = NP // 8
PRX = N // 8
D = 16
H = 32
E = 1600000
NC = 2
NS = 16
EW = E // (NC * NS)
CHUNK = 400
NCHUNK = EW // CHUNK
IDXB = 5
ROWS_T = NP // NS
BLKP = 512


def _lin1p_body(x_ref, w_ref, b_ref, o_ref):
    o_ref[...] = jnp.maximum(
        jnp.dot(x_ref[...], w_ref[...], preferred_element_type=jnp.float32)
        + b_ref[...], 0.0)


def _lin1p(xp, W1b, b1b):
    return pl.pallas_call(
        _lin1p_body,
        grid=(pl.cdiv(PRX, BLKP),),
        in_specs=[pl.BlockSpec((BLKP, 128), lambda i: (i, 0)),
                  pl.BlockSpec((128, 128), lambda i: (0, 0)),
                  pl.BlockSpec((1, 128), lambda i: (0, 0))],
        out_specs=pl.BlockSpec((BLKP, 128), lambda i: (i, 0)),
        out_shape=jax.ShapeDtypeStruct((PRX, 128), jnp.float32),
    )(xp, W1b, b1b)


def _sc_body(h_hbm, src_hbm, dst_hbm, ones_hbm, z2d_hbm, z1d_hbm,
             sum_out, cnt_out,
             acc_sh, cnt_sh, srcb_v, dstb_v, rows_v0, rows_v1,
             ones_v, cb_v, sem0, sem1, semS0, semS1):
    c = lax.axis_index("c")
    s = lax.axis_index("s")
    zstart = s * ROWS_T
    pltpu.sync_copy(z2d_hbm, acc_sh.at[pl.ds(zstart, ROWS_T)])
    pltpu.sync_copy(z1d_hbm, cb_v)
    for off, ln in _SUBCHUNKS:
        pltpu.sync_copy(cb_v.at[pl.ds(0, ln)], cnt_sh.at[pl.ds(zstart + off, ln)])
    pltpu.sync_copy(ones_hbm, ones_v)
    plsc.subcore_barrier()

    rows_b = (rows_v0, rows_v1)
    semG = (sem0, sem1)
    semS = (semS0, semS1)

    def finish(k, last):
        p = k % 2
        pltpu.make_async_copy(h_hbm.at[srcb_v.at[k]], rows_b[p], semG[p]).wait()
        pltpu.sync_copy(ones_v, cnt_sh.at[dstb_v.at[k]], add=True)
        if last:
            pltpu.sync_copy(rows_b[p], acc_sh.at[dstb_v.at[k]], add=True)
        else:
            pltpu.async_copy(rows_b[p], acc_sh.at[dstb_v.at[k]], semS[p], add=True)

    def batch(b, first, last):
        r0 = (c * NS + s) * NCHUNK + b * IDXB
        if not first:
            for p in (0, 1):
                pltpu.make_async_copy(rows_b[p], acc_sh.at[dstb_v.at[p]],
                                      semS[p]).wait()
        pltpu.sync_copy(src_hbm.at[pl.ds(r0, IDXB)], srcb_v)
        pltpu.sync_copy(dst_hbm.at[pl.ds(r0, IDXB)], dstb_v)
        for k in range(IDXB):
            if k >= 2 and not last:
                p = k % 2
                pltpu.make_async_copy(rows_b[p], acc_sh.at[dstb_v.at[k]],
                                      semS[p]).wait()
            pltpu.async_copy(h_hbm.at[srcb_v.at[k]], rows_b[k % 2], semG[k % 2])
            if k > 0:
                finish(k - 1, last)
        finish(IDXB - 1, last)

    batch(0, True, False)
    lax.fori_loop(1, NCHUNK // IDXB - 1, lambda b, cr: (batch(b, False, False), cr)[1], 0)
    batch(NCHUNK // IDXB - 1, False, True)
    plsc.subcore_barrier()
    pltpu.sync_copy(acc_sh.at[pl.ds(zstart, ROWS_T)],
                    sum_out.at[c, pl.ds(zstart, ROWS_T)])
    for off, ln in _SUBCHUNKS:
        pltpu.sync_copy(cnt_sh.at[pl.ds(zstart + off, ln)], cb_v.at[pl.ds(0, ln)])
        pltpu.sync_copy(cb_v.at[pl.ds(0, ln)],
                        cnt_out.at[pl.ds(c * NP + zstart + off, ln)])


CSTAGE = 800
_SUBCHUNKS = tuple((k * CSTAGE, CSTAGE) for k in range(ROWS_T // CSTAGE))


def _sc_aggregate(h_lin, src, dst):
    ones = jnp.ones((CHUNK,), jnp.float32)
    z2d = jnp.zeros((ROWS_T, D), jnp.float32)
    z1d = jnp.zeros((CSTAGE,), jnp.float32)
    mesh = plsc.VectorSubcoreMesh(core_axis_name="c", subcore_axis_name="s")
    f = pl.kernel(
        _sc_body,
        out_type=[jax.ShapeDtypeStruct((NC, NP, D), jnp.float32),
                  jax.ShapeDtypeStruct((NC * NP,), jnp.float32)],
        mesh=mesh,
        scratch_types=[
            pltpu.VMEM_SHARED((NP, D), jnp.float32),
            pltpu.VMEM_SHARED((NP,), jnp.float32),
            pltpu.VMEM((IDXB, CHUNK), jnp.int32),
            pltpu.VMEM((IDXB, CHUNK), jnp.int32),
            pltpu.VMEM((CHUNK, D), jnp.float32),
            pltpu.VMEM((CHUNK, D), jnp.float32),
            pltpu.VMEM((CHUNK,), jnp.float32),
            pltpu.VMEM((CSTAGE,), jnp.float32),
            pltpu.SemaphoreType.DMA,
            pltpu.SemaphoreType.DMA,
            pltpu.SemaphoreType.DMA,
            pltpu.SemaphoreType.DMA,
        ],
        compiler_params=pltpu.CompilerParams(use_tc_tiling_on_sc=False),
    )
    return f(h_lin, src.reshape(E // CHUNK, CHUNK), dst.reshape(E // CHUNK, CHUNK),
             ones, z2d, z1d)


def _combinep_body(h_ref, sp_ref, d_ref,
                   wl_ref, bl_ref, wr_ref, w2_ref, b2_ref, o_ref):
    agg = (sp_ref[0] + sp_ref[1]) / jnp.maximum(d_ref[...], 1.0)
    h2 = jnp.maximum(
        jnp.dot(agg, wl_ref[...], preferred_element_type=jnp.float32)
        + bl_ref[...]
        + jnp.dot(h_ref[...], wr_ref[...], preferred_element_type=jnp.float32),
        0.0)
    o_ref[...] = (jnp.dot(h2, w2_ref[...], preferred_element_type=jnp.float32)
                  + b2_ref[...])


def _combinep(hp, sp, divp, Wlb, blb, Wrb, W2b, b2b):
    return pl.pallas_call(
        _combinep_body,
        grid=(pl.cdiv(PRX, BLKP),),
        in_specs=[pl.BlockSpec((BLKP, 128), lambda i: (i, 0)),
                  pl.BlockSpec((NC, BLKP, 128), lambda i: (0, i, 0)),
                  pl.BlockSpec((BLKP, 128), lambda i: (i, 0)),
                  pl.BlockSpec((128, 256), lambda i: (0, 0)),
                  pl.BlockSpec((1, 256), lambda i: (0, 0)),
                  pl.BlockSpec((128, 256), lambda i: (0, 0)),
                  pl.BlockSpec((256, 256), lambda i: (0, 0)),
                  pl.BlockSpec((1, 256), lambda i: (0, 0))],
        out_specs=pl.BlockSpec((BLKP, 256), lambda i: (i, 0)),
        out_shape=jax.ShapeDtypeStruct((PRX, 256), jnp.float32),
    )(hp, sp, divp, Wlb, blb, Wrb, W2b, b2b)


def _block_diag8(W):
    a, b = W.shape
    eye = jnp.eye(8, dtype=W.dtype)
    return (eye[:, None, :, None] * W[None, :, None, :]).reshape(8 * a, 8 * b)


def kernel(x, edge_index, W1, b1, Wl, bl, Wr, W2, b2):
    ei = edge_index.astype(jnp.int32)
    src = ei[0]
    dst = ei[1]

    W1b = _block_diag8(W1)
    b1b = jnp.tile(b1, 8).reshape(1, 128)
    Wlb = _block_diag8(Wl)
    blb = jnp.tile(bl, 8).reshape(1, 256)
    Wrb = _block_diag8(Wr)
    W2b = _block_diag8(W2)
    b2b = jnp.tile(b2, 8).reshape(1, 256)

    xp = x.reshape(PRX, 128)
    hp = _lin1p(xp, W1b, b1b)
    summed, cnt = _sc_aggregate(hp.reshape(N, D), src, dst)
    sp = summed.reshape(NC, PR, 128)
    cm = cnt[:N] + cnt[NP:NP + N]
    expand = jnp.kron(jnp.eye(8, dtype=jnp.float32), jnp.ones((1, D), jnp.float32))
    divp = cm.reshape(PRX, 8) @ expand
    outp = _combinep(hp, sp, divp, Wlb, blb, Wrb, W2b, b2b)
    return outp.reshape(N, H)


# speedup vs baseline: 1.0002x; 1.0002x over previous
import jax
import jax.numpy as jnp
from jax import lax
from jax.experimental import pallas as pl
from jax.experimental.pallas import tpu as pltpu
from jax.experimental.pallas import tpu_sc as plsc

N = 100000
NP = 102400
PR = NP // 8
PRX = N // 8
D = 16
H = 32
E = 1600000
NC = 2
NS = 16
EW = E // (NC * NS)
CHUNK = 400
NCHUNK = EW // CHUNK
IDXB = 5
ROWS_T = NP // NS
BLKP = 512


def _lin1p_body(x_ref, w_ref, b_ref, o_ref):
    o_ref[...] = jnp.maximum(
        jnp.dot(x_ref[...], w_ref[...], preferred_element_type=jnp.float32)
        + b_ref[...], 0.0)


def _lin1p(xp, W1b, b1b):
    return pl.pallas_call(
        _lin1p_body,
        grid=(pl.cdiv(PRX, BLKP),),
        in_specs=[pl.BlockSpec((BLKP, 128), lambda i: (i, 0)),
                  pl.BlockSpec((128, 128), lambda i: (0, 0)),
                  pl.BlockSpec((1, 128), lambda i: (0, 0))],
        out_specs=pl.BlockSpec((BLKP, 128), lambda i: (i, 0)),
        out_shape=jax.ShapeDtypeStruct((PRX, 128), jnp.float32),
    )(xp, W1b, b1b)


def _sc_body(h_hbm, src_hbm, dst_hbm, ones_hbm, z2d_hbm, z1d_hbm,
             sum_out, cnt_out,
             acc_sh, cnt_sh, srcb_v, dstb_v, rows_v0, rows_v1,
             ones_v, cb_v, sem0, sem1):
    c = lax.axis_index("c")
    s = lax.axis_index("s")
    zstart = s * ROWS_T
    pltpu.sync_copy(z2d_hbm, acc_sh.at[pl.ds(zstart, ROWS_T)])
    pltpu.sync_copy(z1d_hbm, cb_v)
    for off, ln in _SUBCHUNKS:
        pltpu.sync_copy(cb_v.at[pl.ds(0, ln)], cnt_sh.at[pl.ds(zstart + off, ln)])
    pltpu.sync_copy(ones_hbm, ones_v)
    plsc.subcore_barrier()

    rows_b = (rows_v0, rows_v1)
    sems = (sem0, sem1)

    def drain(k):
        p = k % 2
        pltpu.make_async_copy(h_hbm.at[srcb_v.at[k]], rows_b[p], sems[p]).wait()
        pltpu.sync_copy(rows_b[p], acc_sh.at[dstb_v.at[k]], add=True)
        pltpu.sync_copy(ones_v, cnt_sh.at[dstb_v.at[k]], add=True)

    def batch(b, carry):
        r0 = (c * NS + s) * NCHUNK + b * IDXB
        pltpu.sync_copy(src_hbm.at[pl.ds(r0, IDXB)], srcb_v)
        pltpu.sync_copy(dst_hbm.at[pl.ds(r0, IDXB)], dstb_v)
        for k in range(IDXB):
            pltpu.async_copy(h_hbm.at[srcb_v.at[k]], rows_b[k % 2], sems[k % 2])
            if k > 0:
                drain(k - 1)
        drain(IDXB - 1)
        return carry

    lax.fori_loop(0, NCHUNK // IDXB, batch, 0)
    plsc.subcore_barrier()
    pltpu.sync_copy(acc_sh.at[pl.ds(zstart, ROWS_T)],
                    sum_out.at[c, pl.ds(zstart, ROWS_T)])
    for off, ln in _SUBCHUNKS:
        pltpu.sync_copy(cnt_sh.at[pl.ds(zstart + off, ln)], cb_v.at[pl.ds(0, ln)])
        pltpu.sync_copy(cb_v.at[pl.ds(0, ln)],
                        cnt_out.at[pl.ds(c * NP + zstart + off, ln)])


CSTAGE = 800
_SUBCHUNKS = tuple((k * CSTAGE, CSTAGE) for k in range(ROWS_T // CSTAGE))


def _sc_aggregate(h_lin, src, dst):
    ones = jnp.ones((CHUNK,), jnp.float32)
    z2d = jnp.zeros((ROWS_T, D), jnp.float32)
    z1d = jnp.zeros((CSTAGE,), jnp.float32)
    mesh = plsc.VectorSubcoreMesh(core_axis_name="c", subcore_axis_name="s")
    f = pl.kernel(
        _sc_body,
        out_type=[jax.ShapeDtypeStruct((NC, NP, D), jnp.float32),
                  jax.ShapeDtypeStruct((NC * NP,), jnp.float32)],
        mesh=mesh,
        scratch_types=[
            pltpu.VMEM_SHARED((NP, D), jnp.float32),
            pltpu.VMEM_SHARED((NP,), jnp.float32),
            pltpu.VMEM((IDXB, CHUNK), jnp.int32),
            pltpu.VMEM((IDXB, CHUNK), jnp.int32),
            pltpu.VMEM((CHUNK, D), jnp.float32),
            pltpu.VMEM((CHUNK, D), jnp.float32),
            pltpu.VMEM((CHUNK,), jnp.float32),
            pltpu.VMEM((CSTAGE,), jnp.float32),
            pltpu.SemaphoreType.DMA,
            pltpu.SemaphoreType.DMA,
        ],
        compiler_params=pltpu.CompilerParams(use_tc_tiling_on_sc=False),
    )
    return f(h_lin, src.reshape(E // CHUNK, CHUNK), dst.reshape(E // CHUNK, CHUNK),
             ones, z2d, z1d)


def _combinep_body(h_ref, sp_ref, d_ref,
                   wl_ref, bl_ref, wr_ref, w2_ref, b2_ref, o_ref):
    agg = (sp_ref[0] + sp_ref[1]) / jnp.maximum(d_ref[...], 1.0)
    h2 = jnp.maximum(
        jnp.dot(agg, wl_ref[...], preferred_element_type=jnp.float32)
        + bl_ref[...]
        + jnp.dot(h_ref[...], wr_ref[...], preferred_element_type=jnp.float32),
        0.0)
    o_ref[...] = (jnp.dot(h2, w2_ref[...], preferred_element_type=jnp.float32)
                  + b2_ref[...])


def _combinep(hp, sp, divp, Wlb, blb, Wrb, W2b, b2b):
    return pl.pallas_call(
        _combinep_body,
        grid=(pl.cdiv(PRX, BLKP),),
        in_specs=[pl.BlockSpec((BLKP, 128), lambda i: (i, 0)),
                  pl.BlockSpec((NC, BLKP, 128), lambda i: (0, i, 0)),
                  pl.BlockSpec((BLKP, 128), lambda i: (i, 0)),
                  pl.BlockSpec((128, 256), lambda i: (0, 0)),
                  pl.BlockSpec((1, 256), lambda i: (0, 0)),
                  pl.BlockSpec((128, 256), lambda i: (0, 0)),
                  pl.BlockSpec((256, 256), lambda i: (0, 0)),
                  pl.BlockSpec((1, 256), lambda i: (0, 0))],
        out_specs=pl.BlockSpec((BLKP, 256), lambda i: (i, 0)),
        out_shape=jax.ShapeDtypeStruct((PRX, 256), jnp.float32),
    )(hp, sp, divp, Wlb, blb, Wrb, W2b, b2b)


def _block_diag8(W):
    a, b = W.shape
    eye = jnp.eye(8, dtype=W.dtype)
    return (eye[:, None, :, None] * W[None, :, None, :]).reshape(8 * a, 8 * b)


def kernel(x, edge_index, W1, b1, Wl, bl, Wr, W2, b2):
    ei = edge_index.astype(jnp.int32)
    src = ei[0]
    dst = ei[1]

    W1b = _block_diag8(W1)
    b1b = jnp.tile(b1, 8).reshape(1, 128)
    Wlb = _block_diag8(Wl)
    blb = jnp.tile(bl, 8).reshape(1, 256)
    Wrb = _block_diag8(Wr)
    W2b = _block_diag8(W2)
    b2b = jnp.tile(b2, 8).reshape(1, 256)

    xp = x.reshape(PRX, 128)
    hp = _lin1p(xp, W1b, b1b)
    summed, cnt = _sc_aggregate(hp.reshape(N, D), src, dst)
    sp = summed.reshape(NC, PR, 128)
    cm = cnt[:N] + cnt[NP:NP + N]
    expand = jnp.kron(jnp.eye(8, dtype=jnp.float32), jnp.ones((1, D), jnp.float32))
    divp = cm.reshape(PRX, 8) @ expand
    outp = _combinep(hp, sp, divp, Wlb, blb, Wrb, W2b, b2b)
    return outp.reshape(N, H)
